# Initial kernel scaffold; baseline (speedup 1.0000x reference)
import jax
import jax.numpy as jnp
from jax.experimental import pallas as pl


def kernel(unknown, known, unknow_feats, known_feats, W1, b1, g1, be1, W2, b2, g2, be2):
    raise NotImplementedError("write your pallas kernel here")


# speedup vs baseline: 23.1197x; 23.1197x over previous
import functools

import jax
import jax.numpy as jnp
from jax import lax
from jax.experimental import pallas as pl

_TILE = 256


def _stage_a(unknown_ref, known_ref, kf_ref, uf_ref, W1_ref, b1_ref,
             h1_ref, s1_ref, q1_ref):
    b = pl.program_id(0)
    t = pl.program_id(1)
    U = unknown_ref[0]
    K = known_ref[0]
    su2 = jnp.sum(U * U, axis=1, keepdims=True)
    sk2 = jnp.sum(K * K, axis=1)[None, :]
    dot = lax.dot_general(U, K, (((1,), (1,)), ((), ())),
                          preferred_element_type=jnp.float32)
    d2 = su2 - 2.0 * dot + sk2
    tile, m = d2.shape
    col = lax.broadcasted_iota(jnp.int32, (tile, m), 1)
    BIG = jnp.float32(3.0e38)
    d2w = d2
    dists = []
    onehots = []
    for _ in range(3):
        mj = jnp.min(d2w, axis=1, keepdims=True)
        ij = jnp.min(jnp.where(d2w == mj, col, jnp.int32(2147483647)),
                     axis=1, keepdims=True)
        oh = col == ij
        dists.append(mj)
        onehots.append(oh)
        d2w = jnp.where(oh, BIG, d2w)
    recips = [1.0 / (dj + 1e-8) for dj in dists]
    norm = recips[0] + recips[1] + recips[2]
    S = jnp.zeros_like(d2)
    for oh, rj in zip(onehots, recips):
        S = S + jnp.where(oh, rj / norm, 0.0)
    kf = kf_ref[0]
    interpT = lax.dot_general(kf, S, (((0,), (1,)), ((), ())),
                              preferred_element_type=jnp.float32)
    feat = jnp.concatenate([interpT, uf_ref[0]], axis=0)
    h1 = lax.dot_general(W1_ref[...], feat, (((1,), (0,)), ((), ())),
                         preferred_element_type=jnp.float32) + b1_ref[...]
    h1_ref[0] = h1

    @pl.when(jnp.logical_and(b == 0, t == 0))
    def _():
        s1_ref[...] = jnp.zeros_like(s1_ref)
        q1_ref[...] = jnp.zeros_like(q1_ref)

    s1_ref[...] += jnp.sum(h1, axis=1, keepdims=True)
    q1_ref[...] += jnp.sum(h1 * h1, axis=1, keepdims=True)


def _bn_scale_shift(s_ref, q_ref, g_ref, be_ref, N):
    mu = s_ref[...] / N
    var = q_ref[...] / N - mu * mu
    a = g_ref[...] * lax.rsqrt(var + 1e-5)
    c = be_ref[...] - a * mu
    return a, c


def _stage_b(h1_ref, s1_ref, q1_ref, W2_ref, b2_ref, g1_ref, be1_ref,
             s2_ref, q2_ref, *, N):
    b = pl.program_id(0)
    t = pl.program_id(1)
    a1, c1 = _bn_scale_shift(s1_ref, q1_ref, g1_ref, be1_ref, N)
    r = jnp.maximum(a1 * h1_ref[0] + c1, 0.0)
    h2 = lax.dot_general(W2_ref[...], r, (((1,), (0,)), ((), ())),
                         preferred_element_type=jnp.float32) + b2_ref[...]

    @pl.when(jnp.logical_and(b == 0, t == 0))
    def _():
        s2_ref[...] = jnp.zeros_like(s2_ref)
        q2_ref[...] = jnp.zeros_like(q2_ref)

    s2_ref[...] += jnp.sum(h2, axis=1, keepdims=True)
    q2_ref[...] += jnp.sum(h2 * h2, axis=1, keepdims=True)


def _stage_c(h1_ref, s1_ref, q1_ref, s2_ref, q2_ref, W2_ref, b2_ref,
             g1_ref, be1_ref, g2_ref, be2_ref, out_ref, *, N):
    a1, c1 = _bn_scale_shift(s1_ref, q1_ref, g1_ref, be1_ref, N)
    r = jnp.maximum(a1 * h1_ref[0] + c1, 0.0)
    h2 = lax.dot_general(W2_ref[...], r, (((1,), (0,)), ((), ())),
                         preferred_element_type=jnp.float32) + b2_ref[...]
    a2, c2 = _bn_scale_shift(s2_ref, q2_ref, g2_ref, be2_ref, N)
    out_ref[0] = jnp.maximum(a2 * h2 + c2, 0.0)


def kernel(unknown, known, unknow_feats, known_feats,
           W1, b1, g1, be1, W2, b2, g2, be2):
    B, n, _ = unknown.shape
    m = known.shape[1]
    C1 = unknow_feats.shape[1]
    C2 = known_feats.shape[1]
    Cin = C1 + C2
    Co = W1.shape[0]
    tile = min(_TILE, n)
    NT = n // tile
    grid = (B, NT)
    N = float(B * n)

    kfT = jnp.transpose(known_feats, (0, 2, 1))
    b1c, g1c, be1c = b1[:, None], g1[:, None], be1[:, None]
    b2c, g2c, be2c = b2[:, None], g2[:, None], be2[:, None]

    vec = pl.BlockSpec((Co, 1), lambda b, t: (0, 0))
    h1_spec = pl.BlockSpec((1, Co, tile), lambda b, t: (b, 0, t))

    h1, s1, q1 = pl.pallas_call(
        _stage_a,
        grid=grid,
        in_specs=[
            pl.BlockSpec((1, tile, 3), lambda b, t: (b, t, 0)),
            pl.BlockSpec((1, m, 3), lambda b, t: (b, 0, 0)),
            pl.BlockSpec((1, m, C2), lambda b, t: (b, 0, 0)),
            pl.BlockSpec((1, C1, tile), lambda b, t: (b, 0, t)),
            pl.BlockSpec((Co, Cin), lambda b, t: (0, 0)),
            vec,
        ],
        out_specs=[h1_spec, vec, vec],
        out_shape=[
            jax.ShapeDtypeStruct((B, Co, n), jnp.float32),
            jax.ShapeDtypeStruct((Co, 1), jnp.float32),
            jax.ShapeDtypeStruct((Co, 1), jnp.float32),
        ],
    )(unknown, known, kfT, unknow_feats, W1, b1c)

    s2, q2 = pl.pallas_call(
        functools.partial(_stage_b, N=N),
        grid=grid,
        in_specs=[h1_spec, vec, vec,
                  pl.BlockSpec((Co, Co), lambda b, t: (0, 0)),
                  vec, vec, vec],
        out_specs=[vec, vec],
        out_shape=[
            jax.ShapeDtypeStruct((Co, 1), jnp.float32),
            jax.ShapeDtypeStruct((Co, 1), jnp.float32),
        ],
    )(h1, s1, q1, W2, b2c, g1c, be1c)

    out = pl.pallas_call(
        functools.partial(_stage_c, N=N),
        grid=grid,
        in_specs=[h1_spec, vec, vec, vec, vec,
                  pl.BlockSpec((Co, Co), lambda b, t: (0, 0)),
                  vec, vec, vec, vec, vec],
        out_specs=h1_spec,
        out_shape=jax.ShapeDtypeStruct((B, Co, n), jnp.float32),
    )(h1, s1, q1, s2, q2, W2, b2c, g1c, be1c, g2c, be2c)

    return out


# speedup vs baseline: 32.4483x; 1.4035x over previous
import functools

import jax
import jax.numpy as jnp
from jax import lax
from jax.experimental import pallas as pl
from jax.experimental.pallas import tpu as pltpu

_TILE = 256


def _bn_scale_shift(s_ref, q_ref, g_ref, be_ref, N):
    mu = s_ref[...] / N
    var = q_ref[...] / N - mu * mu
    a = g_ref[...] * lax.rsqrt(var + 1e-5)
    c = be_ref[...] - a * mu
    return a, c


def _fused(unknown_ref, known_ref, kf_ref, uf_ref, W1_ref, b1_ref, W2_ref,
           b2_ref, g1_ref, be1_ref, g2_ref, be2_ref, out_ref,
           h1_ref, h2_ref, s1_ref, q1_ref, s2_ref, q2_ref, *, NT, N):
    p = pl.program_id(0)
    b = pl.program_id(1)
    t = pl.program_id(2)
    tile = out_ref.shape[2]
    col0 = (b * NT + t) * tile
    first = jnp.logical_and(b == 0, t == 0)

    @pl.when(p == 0)
    def _phase0():
        U = unknown_ref[0]
        K = known_ref[0]
        su2 = jnp.sum(U * U, axis=1, keepdims=True)
        sk2 = jnp.sum(K * K, axis=1)[None, :]
        dot = lax.dot_general(U, K, (((1,), (1,)), ((), ())),
                              preferred_element_type=jnp.float32)
        d2 = su2 - 2.0 * dot + sk2
        BIG = jnp.float32(3.0e38)
        m1 = jnp.min(d2, axis=1, keepdims=True)
        m2 = jnp.min(jnp.where(d2 <= m1, BIG, d2), axis=1, keepdims=True)
        m3 = jnp.min(jnp.where(d2 <= m2, BIG, d2), axis=1, keepdims=True)
        r1 = 1.0 / (m1 + 1e-8)
        r2 = 1.0 / (m2 + 1e-8)
        r3 = 1.0 / (m3 + 1e-8)
        norm = r1 + r2 + r3
        S = jnp.where(d2 == m1, r1 / norm,
                      jnp.where(d2 == m2, r2 / norm,
                                jnp.where(d2 == m3, r3 / norm, 0.0)))
        kf = kf_ref[0]
        interpT = lax.dot_general(kf, S, (((0,), (1,)), ((), ())),
                                  preferred_element_type=jnp.float32)
        feat = jnp.concatenate([interpT, uf_ref[0]], axis=0)
        h1 = lax.dot_general(W1_ref[...], feat, (((1,), (0,)), ((), ())),
                             preferred_element_type=jnp.float32) + b1_ref[...]
        h1_ref[:, pl.ds(col0, tile)] = h1

        @pl.when(first)
        def _():
            s1_ref[...] = jnp.zeros_like(s1_ref)
            q1_ref[...] = jnp.zeros_like(q1_ref)

        s1_ref[...] += jnp.sum(h1, axis=1, keepdims=True)
        q1_ref[...] += jnp.sum(h1 * h1, axis=1, keepdims=True)

    @pl.when(p == 1)
    def _phase1():
        a1, c1 = _bn_scale_shift(s1_ref, q1_ref, g1_ref, be1_ref, N)
        r = jnp.maximum(a1 * h1_ref[:, pl.ds(col0, tile)] + c1, 0.0)
        h2 = lax.dot_general(W2_ref[...], r, (((1,), (0,)), ((), ())),
                             preferred_element_type=jnp.float32) + b2_ref[...]
        h2_ref[:, pl.ds(col0, tile)] = h2

        @pl.when(first)
        def _():
            s2_ref[...] = jnp.zeros_like(s2_ref)
            q2_ref[...] = jnp.zeros_like(q2_ref)

        s2_ref[...] += jnp.sum(h2, axis=1, keepdims=True)
        q2_ref[...] += jnp.sum(h2 * h2, axis=1, keepdims=True)

    @pl.when(p == 2)
    def _phase2():
        a2, c2 = _bn_scale_shift(s2_ref, q2_ref, g2_ref, be2_ref, N)
        out_ref[0] = jnp.maximum(a2 * h2_ref[:, pl.ds(col0, tile)] + c2, 0.0)


def kernel(unknown, known, unknow_feats, known_feats,
           W1, b1, g1, be1, W2, b2, g2, be2):
    B, n, _ = unknown.shape
    m = known.shape[1]
    C1 = unknow_feats.shape[1]
    C2 = known_feats.shape[1]
    Cin = C1 + C2
    Co = W1.shape[0]
    tile = min(_TILE, n)
    NT = n // tile
    N = float(B * n)

    kfT = jnp.transpose(known_feats, (0, 2, 1))
    b1c, g1c, be1c = b1[:, None], g1[:, None], be1[:, None]
    b2c, g2c, be2c = b2[:, None], g2[:, None], be2[:, None]

    def p0_map(p, b, t):
        z = (p == 0).astype(jnp.int32)
        return (z * b, 0, z * t)

    vec = pl.BlockSpec((Co, 1), lambda p, b, t: (0, 0))

    out = pl.pallas_call(
        functools.partial(_fused, NT=NT, N=N),
        grid=(3, B, NT),
        in_specs=[
            pl.BlockSpec((1, tile, 3),
                         lambda p, b, t: ((p == 0) * b, (p == 0) * t, 0)),
            pl.BlockSpec((1, m, 3), lambda p, b, t: ((p == 0) * b, 0, 0)),
            pl.BlockSpec((1, m, C2), lambda p, b, t: ((p == 0) * b, 0, 0)),
            pl.BlockSpec((1, C1, tile),
                         lambda p, b, t: ((p == 0) * b, 0, (p == 0) * t)),
            pl.BlockSpec((Co, Cin), lambda p, b, t: (0, 0)),
            vec,
            pl.BlockSpec((Co, Co), lambda p, b, t: (0, 0)),
            vec, vec, vec, vec, vec,
        ],
        out_specs=pl.BlockSpec((1, Co, tile),
                               lambda p, b, t: ((p == 2) * b, 0, (p == 2) * t)),
        out_shape=jax.ShapeDtypeStruct((B, Co, n), jnp.float32),
        scratch_shapes=[
            pltpu.VMEM((Co, B * n), jnp.float32),
            pltpu.VMEM((Co, B * n), jnp.float32),
            pltpu.VMEM((Co, 1), jnp.float32),
            pltpu.VMEM((Co, 1), jnp.float32),
            pltpu.VMEM((Co, 1), jnp.float32),
            pltpu.VMEM((Co, 1), jnp.float32),
        ],
    )(unknown, known, kfT, unknow_feats, W1, b1c, W2, b2c,
      g1c, be1c, g2c, be2c)

    return out
